# baseline (speedup 1.0000x reference)
import functools

import jax
import jax.numpy as jnp
from jax import lax
from jax.experimental import pallas as pl
from jax.experimental.pallas import tpu as pltpu
from jax.experimental.pallas import tpu_sc as plsc

B = 1024
D = 16
V = 100000
G = 8 * D

_NC = 2
_NS = 16
_NW = _NC * _NS
_B_PER_W = B // _NW


def _gather_groups(table, idx):
    mesh = plsc.VectorSubcoreMesh(core_axis_name="c", subcore_axis_name="s")

    @functools.partial(
        pl.kernel,
        mesh=mesh,
        out_type=jax.ShapeDtypeStruct((B, G), table.dtype),
        scratch_types=[
            pltpu.VMEM((_B_PER_W,), jnp.int32),
            pltpu.VMEM((_B_PER_W, G), table.dtype),
            pltpu.SemaphoreType.DMA,
        ],
    )
    def k(table_hbm, idx_hbm, out_hbm, idx_v, rows_v, sem):
        wid = lax.axis_index("s") * _NC + lax.axis_index("c")
        base = wid * _B_PER_W
        pltpu.sync_copy(idx_hbm.at[pl.ds(base, _B_PER_W)], idx_v)
        pltpu.async_copy(table_hbm.at[idx_v], rows_v, sem).wait()
        pltpu.sync_copy(rows_v, out_hbm.at[pl.ds(base, _B_PER_W)])

    return k(table, idx)


_VB = 4096


def _matmul_kernel(g_ref, r_ref, w_ref, out_ref):
    grp = g_ref[...]
    r = r_ref[...]
    emb = jnp.zeros((B, D), jnp.float32)
    for j in range(8):
        emb = emb + jnp.where(r == j, grp[:, j * D:(j + 1) * D], 0.0)
    out_ref[...] = lax.dot_general(
        emb.astype(jnp.bfloat16),
        w_ref[...],
        dimension_numbers=(((1,), (0,)), ((), ())),
        preferred_element_type=jnp.float32,
    )


def _logits(groups, r, w_t_bf16):
    grid = (V + _VB - 1) // _VB
    return pl.pallas_call(
        _matmul_kernel,
        grid=(grid,),
        in_specs=[
            pl.BlockSpec((B, G), lambda i: (0, 0)),
            pl.BlockSpec((B, 1), lambda i: (0, 0)),
            pl.BlockSpec((D, _VB), lambda i: (0, i)),
        ],
        out_specs=pl.BlockSpec((B, _VB), lambda i: (0, i)),
        out_shape=jax.ShapeDtypeStruct((B, V), jnp.float32),
        compiler_params=pltpu.CompilerParams(
            dimension_semantics=("parallel",),
        ),
    )(groups, r, w_t_bf16)


def kernel(x, W_in, W_out):
    idx = x.astype(jnp.int32)
    table = W_in.reshape(V // 8, G)
    groups = _gather_groups(table, idx >> 3)
    w_t = W_out.T.astype(jnp.bfloat16)
    return _logits(groups, (idx & 7).reshape(B, 1), w_t)
